# baseline (speedup 1.0000x reference)
import jax
import jax.numpy as jnp
from jax.experimental import pallas as pl
from jax.experimental.pallas import tpu as pltpu

_FEAT = 128
_W0_OFF, _W1_OFF, _W2_OFF, _BIAS_OFF = 0, 32, 160, 288
_LR, _LE = 20, 4
_TAIL = 32


def _mlp_kernel(inv_ref, reg_ref, exp_ref, w0p_ref, slab_ref, out_ref):
    f32 = jnp.float32
    bm = inv_ref.shape[0]
    bias = slab_ref[_BIAS_OFF:_BIAS_OFF + 8, :]
    x_inv = inv_ref[...].reshape(bm, _TAIL)
    x_reg = reg_ref[...].reshape(bm, _TAIL)
    x_exp = exp_ref[...].reshape(bm, _TAIL)
    h = jnp.dot(x_inv, w0p_ref[0:_TAIL, :], preferred_element_type=f32)
    h = h + jnp.dot(x_reg, w0p_ref[_TAIL:2 * _TAIL, :],
                    preferred_element_type=f32)
    h = h + jnp.dot(x_exp, w0p_ref[2 * _TAIL:3 * _TAIL, :],
                    preferred_element_type=f32)
    h = jnp.maximum(h + bias[0:1, :], 0.0)
    h = jnp.dot(h, slab_ref[_W1_OFF:_W1_OFF + _FEAT, :],
                preferred_element_type=f32)
    h = jnp.maximum(h + bias[1:2, :], 0.0)
    h = jnp.dot(h, slab_ref[_W2_OFF:_W2_OFF + _FEAT, 0:8],
                preferred_element_type=f32)
    h = jnp.maximum(h + bias[2:3, 0:8], 0.0)
    out_ref[...] = jnp.floor(h)


def _pick_bm(b):
    for bm in (2048, 1024, 512, 256, 128, 64, 32, 16, 8):
        if b % bm == 0:
            return bm
    return b


@jax.jit
def _run(slab, inv, reg, exp):
    B = inv.shape[0]
    nt = inv.shape[1]
    w0p = jnp.zeros((3 * _TAIL, _FEAT), jnp.float32)
    w0p = w0p.at[_TAIL - 1, :].set(slab[_W0_OFF, :])
    w0p = w0p.at[2 * _TAIL - _LR:2 * _TAIL, :].set(
        slab[_W0_OFF + 1:_W0_OFF + 1 + _LR, :])
    w0p = w0p.at[3 * _TAIL - _LE:3 * _TAIL, :].set(
        slab[_W0_OFF + 1 + _LR:_W0_OFF + 1 + _LR + _LE, :])

    bm = _pick_bm(B)
    nb = B // bm
    n_rows = slab.shape[0]
    tail_spec = pl.BlockSpec((bm, 1, 1, _TAIL), lambda i: (i, nt - 1, 0, 0))
    out = pl.pallas_call(
        _mlp_kernel,
        out_shape=jax.ShapeDtypeStruct((B, 8), jnp.float32),
        grid=(nb,),
        in_specs=[
            tail_spec,
            tail_spec,
            tail_spec,
            pl.BlockSpec((3 * _TAIL, _FEAT), lambda i: (0, 0)),
            pl.BlockSpec((n_rows, _FEAT), lambda i: (0, 0)),
        ],
        out_specs=pl.BlockSpec((bm, 8), lambda i: (i, 0)),
        compiler_params=pltpu.CompilerParams(
            dimension_semantics=("parallel",)),
    )(inv, reg, exp, w0p, slab)
    return out


def kernel(slab, past_demands, past_inventories, past_regular_orders,
           past_expedited_orders, past_costs):
    del past_demands, past_costs
    inv = jnp.asarray(past_inventories, jnp.float32)
    reg = jnp.asarray(past_regular_orders, jnp.float32)
    exp = jnp.asarray(past_expedited_orders, jnp.float32)
    B, T = inv.shape[0], inv.shape[1]
    inv, reg, exp = (a.reshape(B, T) for a in (inv, reg, exp))
    if T % _TAIL != 0:
        pad = _TAIL - T % _TAIL
        inv, reg, exp = (jnp.pad(a, ((0, 0), (pad, 0)))
                         for a in (inv, reg, exp))
        T += pad
    nt = T // _TAIL
    inv, reg, exp = (a.reshape(B, nt, 1, _TAIL) for a in (inv, reg, exp))
    out = _run(slab, inv, reg, exp)
    return out[:, 0:1], out[:, 1:2], None


# speedup vs baseline: 5.7289x; 5.7289x over previous
import jax
import jax.numpy as jnp
from jax.experimental import pallas as pl
from jax.experimental.pallas import tpu as pltpu

_FEAT = 128
_W0_OFF, _W1_OFF, _W2_OFF, _BIAS_OFF = 0, 32, 160, 288
_LR, _LE = 20, 4


def _mlp_kernel(inv_ref, reg_ref, exp_ref, w0p_ref, slab_ref, out_ref, *, t):
    f32 = jnp.float32
    bias = slab_ref[_BIAS_OFF:_BIAS_OFF + 8, :]
    h = jnp.dot(inv_ref[...], w0p_ref[0:t, :], preferred_element_type=f32)
    h = h + jnp.dot(reg_ref[...], w0p_ref[t:2 * t, :],
                    preferred_element_type=f32)
    h = h + jnp.dot(exp_ref[...], w0p_ref[2 * t:3 * t, :],
                    preferred_element_type=f32)
    h = jnp.maximum(h + bias[0:1, :], 0.0)
    h = jnp.dot(h, slab_ref[_W1_OFF:_W1_OFF + _FEAT, :],
                preferred_element_type=f32)
    h = jnp.maximum(h + bias[1:2, :], 0.0)
    h = jnp.dot(h, slab_ref[_W2_OFF:_W2_OFF + _FEAT, 0:8],
                preferred_element_type=f32)
    h = jnp.maximum(h + bias[2:3, 0:8], 0.0)
    out_ref[...] = jnp.floor(h)


def _pick_bm(b):
    for bm in (1024, 512, 256, 128, 64, 32, 16, 8):
        if b % bm == 0:
            return bm
    return b


@jax.jit
def _run(slab, inv, reg, exp):
    B, T = inv.shape
    w0p = jnp.zeros((3 * T, _FEAT), jnp.float32)
    w0p = w0p.at[T - 1, :].set(slab[_W0_OFF, :])
    w0p = w0p.at[2 * T - _LR:2 * T, :].set(
        slab[_W0_OFF + 1:_W0_OFF + 1 + _LR, :])
    w0p = w0p.at[3 * T - _LE:3 * T, :].set(
        slab[_W0_OFF + 1 + _LR:_W0_OFF + 1 + _LR + _LE, :])

    bm = _pick_bm(B)
    nb = B // bm
    n_rows = slab.shape[0]
    kern = lambda *refs: _mlp_kernel(*refs, t=T)
    out = pl.pallas_call(
        kern,
        out_shape=jax.ShapeDtypeStruct((B, 8), jnp.float32),
        grid=(nb,),
        in_specs=[
            pl.BlockSpec((bm, T), lambda i: (i, 0)),
            pl.BlockSpec((bm, T), lambda i: (i, 0)),
            pl.BlockSpec((bm, T), lambda i: (i, 0)),
            pl.BlockSpec((3 * T, _FEAT), lambda i: (0, 0)),
            pl.BlockSpec((n_rows, _FEAT), lambda i: (0, 0)),
        ],
        out_specs=pl.BlockSpec((bm, 8), lambda i: (i, 0)),
        compiler_params=pltpu.CompilerParams(
            dimension_semantics=("parallel",)),
    )(inv, reg, exp, w0p, slab)
    return out


def kernel(slab, past_demands, past_inventories, past_regular_orders,
           past_expedited_orders, past_costs):
    del past_demands, past_costs
    inv = jnp.asarray(past_inventories, jnp.float32)
    reg = jnp.asarray(past_regular_orders, jnp.float32)
    exp = jnp.asarray(past_expedited_orders, jnp.float32)
    B, T = inv.shape[0], inv.shape[1]
    out = _run(slab, inv.reshape(B, T), reg.reshape(B, T), exp.reshape(B, T))
    return out[:, 0:1], out[:, 1:2], None


# speedup vs baseline: 6.5567x; 1.1445x over previous
import jax
import jax.numpy as jnp
from jax.experimental import pallas as pl
from jax.experimental.pallas import tpu as pltpu

_FEAT = 128
_W0_OFF, _W1_OFF, _W2_OFF, _BIAS_OFF = 0, 32, 160, 288
_LR, _LE = 20, 4


def _mlp_kernel(inv_ref, reg_ref, exp_ref, w0p_ref, slab_ref, out_ref, *, t):
    f32 = jnp.float32
    bias = slab_ref[_BIAS_OFF:_BIAS_OFF + 8, :]
    h = jnp.dot(inv_ref[...], w0p_ref[0:t, :], preferred_element_type=f32)
    h = h + jnp.dot(reg_ref[...], w0p_ref[t:2 * t, :],
                    preferred_element_type=f32)
    h = h + jnp.dot(exp_ref[...], w0p_ref[2 * t:3 * t, :],
                    preferred_element_type=f32)
    h = jnp.maximum(h + bias[0:1, :], 0.0)
    h = jnp.dot(h, slab_ref[_W1_OFF:_W1_OFF + _FEAT, :],
                preferred_element_type=f32)
    h = jnp.maximum(h + bias[1:2, :], 0.0)
    h = jnp.dot(h, slab_ref[_W2_OFF:_W2_OFF + _FEAT, 0:8],
                preferred_element_type=f32)
    h = jnp.maximum(h + bias[2:3, 0:8], 0.0)
    out_ref[...] = jnp.floor(h)


def _pick_bm(b):
    for bm in (4096, 2048, 1024, 512, 256, 128, 64, 32, 16, 8):
        if b % bm == 0:
            return bm
    return b


@jax.jit
def _run(slab, inv, reg, exp):
    B, T = inv.shape
    w0p = jnp.zeros((3 * T, _FEAT), jnp.float32)
    w0p = w0p.at[T - 1, :].set(slab[_W0_OFF, :])
    w0p = w0p.at[2 * T - _LR:2 * T, :].set(
        slab[_W0_OFF + 1:_W0_OFF + 1 + _LR, :])
    w0p = w0p.at[3 * T - _LE:3 * T, :].set(
        slab[_W0_OFF + 1 + _LR:_W0_OFF + 1 + _LR + _LE, :])

    bm = _pick_bm(B)
    nb = B // bm
    n_rows = slab.shape[0]
    kern = lambda *refs: _mlp_kernel(*refs, t=T)
    out = pl.pallas_call(
        kern,
        out_shape=jax.ShapeDtypeStruct((B, 8), jnp.float32),
        grid=(nb,),
        in_specs=[
            pl.BlockSpec((bm, T), lambda i: (i, 0)),
            pl.BlockSpec((bm, T), lambda i: (i, 0)),
            pl.BlockSpec((bm, T), lambda i: (i, 0)),
            pl.BlockSpec((3 * T, _FEAT), lambda i: (0, 0)),
            pl.BlockSpec((n_rows, _FEAT), lambda i: (0, 0)),
        ],
        out_specs=pl.BlockSpec((bm, 8), lambda i: (i, 0)),
        compiler_params=pltpu.CompilerParams(
            dimension_semantics=("parallel",)),
    )(inv, reg, exp, w0p, slab)
    return out


def kernel(slab, past_demands, past_inventories, past_regular_orders,
           past_expedited_orders, past_costs):
    del past_demands, past_costs
    inv = jnp.asarray(past_inventories, jnp.float32)
    reg = jnp.asarray(past_regular_orders, jnp.float32)
    exp = jnp.asarray(past_expedited_orders, jnp.float32)
    B, T = inv.shape[0], inv.shape[1]
    out = _run(slab, inv.reshape(B, T), reg.reshape(B, T), exp.reshape(B, T))
    return out[:, 0:1], out[:, 1:2], None
